# Initial kernel scaffold; baseline (speedup 1.0000x reference)
import jax
import jax.numpy as jnp
from jax.experimental import pallas as pl


def kernel(x, w, gamma, beta):
    raise NotImplementedError("write your pallas kernel here")


# speedup vs baseline: 1.1330x; 1.1330x over previous
import jax
import jax.numpy as jnp
from jax.experimental import pallas as pl
from jax.experimental.pallas import tpu as pltpu

_EPS = 1e-5


def _conv_pass_kernel(x_ref, w_ref, conv_ref, stats_ref):
    x = x_ref[0].astype(jnp.bfloat16)
    c_in, length = x.shape
    zero_col = jnp.zeros((c_in, 1), dtype=jnp.bfloat16)
    x_m1 = jnp.concatenate([zero_col, x[:, : length - 1]], axis=1)
    x_p1 = jnp.concatenate([x[:, 1:], zero_col], axis=1)
    act = jnp.concatenate([x_m1, x, x_p1], axis=0)
    conv = jnp.dot(w_ref[...], act, preferred_element_type=jnp.float32)
    conv_ref[0] = conv.astype(jnp.bfloat16)
    s = jnp.sum(conv, axis=1, keepdims=True)
    s2 = jnp.sum(conv * conv, axis=1, keepdims=True)
    stats_ref[0] = jnp.concatenate([s, s2], axis=1)


def _bn_relu_kernel(conv_ref, ss_ref, o_ref):
    conv = conv_ref[0].astype(jnp.float32)
    scale = ss_ref[:, 0:1]
    shift = ss_ref[:, 1:2]
    o_ref[0] = jnp.maximum(conv * scale + shift, 0.0)


def kernel(x, w, gamma, beta):
    N, C, L = x.shape

    w2 = jnp.transpose(w, (0, 2, 1)).reshape(C, 3 * C).astype(jnp.bfloat16)
    x = x.astype(jnp.float32)

    conv, partial = pl.pallas_call(
        _conv_pass_kernel,
        grid=(N,),
        in_specs=[
            pl.BlockSpec((1, C, L), lambda n: (n, 0, 0)),
            pl.BlockSpec((C, 3 * C), lambda n: (0, 0)),
        ],
        out_specs=[
            pl.BlockSpec((1, C, L), lambda n: (n, 0, 0)),
            pl.BlockSpec((1, C, 2), lambda n: (n, 0, 0)),
        ],
        out_shape=[
            jax.ShapeDtypeStruct((N, C, L), jnp.bfloat16),
            jax.ShapeDtypeStruct((N, C, 2), jnp.float32),
        ],
        compiler_params=pltpu.CompilerParams(
            dimension_semantics=("parallel",),
        ),
    )(x, w2)

    stats = jnp.sum(partial, axis=0)
    inv_count = 1.0 / float(N * L)
    mean = stats[:, 0] * inv_count
    var = stats[:, 1] * inv_count - mean * mean
    scale = gamma.astype(jnp.float32) * jax.lax.rsqrt(var + _EPS)
    shift = beta.astype(jnp.float32) - mean * scale
    ss = jnp.stack([scale, shift], axis=1)

    out = pl.pallas_call(
        _bn_relu_kernel,
        grid=(N,),
        in_specs=[
            pl.BlockSpec((1, C, L), lambda n: (n, 0, 0)),
            pl.BlockSpec((C, 2), lambda n: (0, 0)),
        ],
        out_specs=pl.BlockSpec((1, C, L), lambda n: (n, 0, 0)),
        out_shape=jax.ShapeDtypeStruct((N, C, L), jnp.float32),
        compiler_params=pltpu.CompilerParams(
            dimension_semantics=("parallel",),
        ),
    )(conv, ss)
    return out


# speedup vs baseline: 1.1828x; 1.0440x over previous
import jax
import jax.numpy as jnp
from jax.experimental import pallas as pl
from jax.experimental.pallas import tpu as pltpu

_EPS = 1e-5


def kernel(x, w, gamma, beta):
    N, C, L = x.shape

    w2 = jnp.transpose(w, (0, 2, 1)).reshape(C, 3 * C).astype(jnp.bfloat16)
    gb = jnp.stack(
        [gamma.astype(jnp.float32), beta.astype(jnp.float32)], axis=1
    )
    x = x.astype(jnp.float32)

    mesh = pltpu.create_tensorcore_mesh("core")
    num_cores = len(mesh.devices)
    npc = N // num_cores
    inv_count = 1.0 / float(N * L)

    pstats0 = jnp.zeros((num_cores, C, 2), jnp.float32)
    out0 = jnp.zeros((N, C, L), jnp.float32)

    def run(refs):
        x_ref, w_ref, gb_ref, pstats_ref, out_ref = refs

        @pl.core_map(mesh)
        def _():
            core_id = jax.lax.axis_index("core")

            def scoped(w_vmem, gb_vmem, conv_vmem, stats_vmem,
                       pstats_vmem, load_sem, store_sem, barrier_sem):
                w_copy = pltpu.make_async_copy(w_ref, w_vmem, load_sem)
                w_copy.start()
                gb_copy = pltpu.make_async_copy(gb_ref, gb_vmem, load_sem)
                gb_copy.start()
                w_copy.wait()
                gb_copy.wait()
                stats_vmem[...] = jnp.zeros_like(stats_vmem)

                def p1_body(x_blk_ref):
                    n = pl.program_id(0)
                    local = n - core_id * npc
                    xb = x_blk_ref[0].astype(jnp.bfloat16)
                    zero_col = jnp.zeros((C, 1), dtype=jnp.bfloat16)
                    x_m1 = jnp.concatenate([zero_col, xb[:, : L - 1]], axis=1)
                    x_p1 = jnp.concatenate([xb[:, 1:], zero_col], axis=1)
                    act = jnp.concatenate([x_m1, xb, x_p1], axis=0)
                    conv = jnp.dot(
                        w_vmem[...], act, preferred_element_type=jnp.float32
                    )
                    conv_vmem[local] = conv.astype(jnp.bfloat16)
                    s = jnp.sum(conv, axis=1, keepdims=True)
                    s2 = jnp.sum(conv * conv, axis=1, keepdims=True)
                    stats_vmem[...] += jnp.concatenate([s, s2], axis=1)

                pltpu.emit_pipeline(
                    p1_body,
                    grid=(N,),
                    in_specs=[pl.BlockSpec((1, C, L), lambda n: (n, 0, 0))],
                    core_axis_name="core",
                    dimension_semantics=(pltpu.PARALLEL,),
                )(x_ref)

                s_copy = pltpu.make_async_copy(
                    stats_vmem, pstats_ref.at[core_id], store_sem
                )
                s_copy.start()
                s_copy.wait()

                for i in range(num_cores):
                    @pl.when(core_id != i)
                    def _():
                        pl.semaphore_signal(barrier_sem, 1, core_index=i)
                pl.semaphore_wait(barrier_sem, num_cores - 1)

                all_copy = pltpu.make_async_copy(
                    pstats_ref, pstats_vmem, load_sem
                )
                all_copy.start()
                all_copy.wait()

                stats = jnp.sum(pstats_vmem[...], axis=0)
                mean = stats[:, 0:1] * inv_count
                var = stats[:, 1:2] * inv_count - mean * mean
                scale = gb_vmem[:, 0:1] * jax.lax.rsqrt(var + _EPS)
                shift = gb_vmem[:, 1:2] - mean * scale

                def p2_body(o_blk_ref):
                    n = pl.program_id(0)
                    local = n - core_id * npc
                    conv = conv_vmem[local].astype(jnp.float32)
                    o_blk_ref[0] = jnp.maximum(conv * scale + shift, 0.0)

                pltpu.emit_pipeline(
                    p2_body,
                    grid=(N,),
                    out_specs=[pl.BlockSpec((1, C, L), lambda n: (n, 0, 0))],
                    core_axis_name="core",
                    dimension_semantics=(pltpu.PARALLEL,),
                )(out_ref)

            pl.run_scoped(
                scoped,
                pltpu.VMEM((C, 3 * C), jnp.bfloat16),
                pltpu.VMEM((C, 2), jnp.float32),
                pltpu.VMEM((npc, C, L), jnp.bfloat16),
                pltpu.VMEM((C, 2), jnp.float32),
                pltpu.VMEM((num_cores, C, 2), jnp.float32),
                pltpu.SemaphoreType.DMA,
                pltpu.SemaphoreType.DMA,
                pltpu.SemaphoreType.REGULAR,
            )

    _, _, _, _, out = pl.run_state(run)((x, w2, gb, pstats0, out0))
    return out
